# baseline (speedup 1.0000x reference)
import functools

import jax
import jax.numpy as jnp
from jax import lax
from jax.experimental import pallas as pl
from jax.experimental.pallas import tpu as pltpu
from jax.experimental.pallas import tpu_sc as plsc

NC = 2
NS = 16
CH = 128


def _pad_n(n):
    q = NS * 8
    return ((n + q) // q) * q


def _pad_e(e):
    q = NC * NS * CH * 8
    return ((e + q - 1) // q) * q


def _elu(v):
    return jnp.where(v > 0, v, jnp.exp(jnp.minimum(v, 0.0)) - 1.0)


def _lrelu(v):
    return jnp.where(v >= 0, v, 0.01 * v)


@functools.lru_cache(maxsize=None)
def _make_sc_degree(n, e):
    per_sub = e // (NC * NS)
    n_chunks = per_sub // CH
    npad = _pad_n(n)
    assert per_sub % CH == 0 and n_chunks % 8 == 0
    rows_per_sub = npad // NS
    mesh = plsc.VectorSubcoreMesh(core_axis_name="c", subcore_axis_name="s")

    @functools.partial(
        pl.kernel,
        mesh=mesh,
        out_type=jax.ShapeDtypeStruct((NC, npad, 128), jnp.float32),
        scratch_types=[
            pltpu.VMEM((n_chunks, CH), jnp.int32),
            pltpu.VMEM((CH, 128), jnp.float32),
            pltpu.VMEM_SHARED((npad, 128), jnp.float32),
        ],
    )
    def deg_kernel(zeros_hbm, dst_hbm, out_hbm, didx, ones_v, acc_sh):
        cid = lax.axis_index("c")
        sid = lax.axis_index("s")
        wrow = (sid * NC + cid) * n_chunks
        pltpu.sync_copy(dst_hbm.at[pl.ds(wrow, n_chunks)], didx)

        @pl.loop(0, CH)
        def _(i):
            @pl.loop(0, 128, step=16)
            def _(j):
                ones_v[i, pl.ds(j, 16)] = jnp.ones((16,), jnp.float32)

        pltpu.sync_copy(zeros_hbm, acc_sh.at[pl.ds(sid * rows_per_sub, rows_per_sub)])
        plsc.subcore_barrier()

        @pl.loop(0, n_chunks)
        def _(k):
            pltpu.sync_copy(ones_v, acc_sh.at[didx.at[k]], add=True)

        plsc.subcore_barrier()
        rs = pl.ds(sid * rows_per_sub, rows_per_sub)
        pltpu.sync_copy(acc_sh.at[rs], out_hbm.at[cid, rs])

    return deg_kernel


@functools.lru_cache(maxsize=None)
def _make_sc_agg(n, e, d):
    per_sub = e // (NC * NS)
    n_chunks = per_sub // CH
    npad = _pad_n(n)
    nblk = n_chunks // 8
    assert per_sub % CH == 0 and n_chunks % 16 == 0 and nblk >= 2
    rows_per_sub = npad // NS
    mesh = plsc.VectorSubcoreMesh(core_axis_name="c", subcore_axis_name="s")

    @functools.partial(
        pl.kernel,
        mesh=mesh,
        out_type=jax.ShapeDtypeStruct((NC, npad, d), jnp.float32),
        scratch_types=[
            pltpu.VMEM((n_chunks, CH), jnp.int32),
            pltpu.VMEM((8, CH), jnp.int32),
            pltpu.VMEM((8, CH), jnp.int32),
            pltpu.VMEM((CH, d), jnp.float32),
            pltpu.VMEM((CH, d), jnp.float32),
            pltpu.SemaphoreType.DMA,
            pltpu.SemaphoreType.DMA,
            pltpu.SemaphoreType.DMA,
            pltpu.VMEM_SHARED((npad, d), jnp.float32),
        ],
    )
    def agg_kernel(zeros_hbm, h_hbm, src_hbm, dst_hbm, out_hbm,
                   didx, sblk0, sblk1, buf0, buf1, g0, g1, isem, acc_sh):
        sblks = (sblk0, sblk1)
        bufs = (buf0, buf1)
        gsems = (g0, g1)
        cid = lax.axis_index("c")
        sid = lax.axis_index("s")
        wrow = (sid * NC + cid) * n_chunks
        pltpu.sync_copy(dst_hbm.at[pl.ds(wrow, n_chunks)], didx)
        pltpu.sync_copy(src_hbm.at[pl.ds(wrow, 8)], sblk0)
        pltpu.sync_copy(src_hbm.at[pl.ds(wrow + 8, 8)], sblk1)
        pltpu.sync_copy(zeros_hbm, acc_sh.at[pl.ds(sid * rows_per_sub, rows_per_sub)])
        plsc.subcore_barrier()

        def fire(sb, r, b):
            pltpu.async_copy(h_hbm.at[sblks[sb].at[r]], bufs[b], gsems[b])

        def wait_g(sb, r, b):
            pltpu.make_async_copy(h_hbm.at[sblks[sb].at[r]], bufs[b], gsems[b]).wait()

        def idx_cp(j, sb):
            return pltpu.make_async_copy(
                src_hbm.at[pl.ds(wrow + j * 8, 8)], sblks[sb], isem)

        @pl.loop(0, nblk, step=2)
        def _(j):
            c0 = j * 8

            @pl.when(j > 0)
            def _():
                pltpu.sync_copy(src_hbm.at[pl.ds(wrow + j * 8, 8)], sblk0)
                pltpu.sync_copy(src_hbm.at[pl.ds(wrow + (j + 1) * 8, 8)], sblk1)

            fire(0, 0, 0)
            fire(0, 1, 1)
            for t in range(16):
                sb, r = divmod(t, 8)
                b = t % 2
                wait_g(sb, r, b)
                pltpu.sync_copy(bufs[b], acc_sh.at[didx.at[c0 + t]], add=True)
                if t < 14:
                    sb2, r2 = divmod(t + 2, 8)
                    fire(sb2, r2, b)

        plsc.subcore_barrier()
        rs = pl.ds(sid * rows_per_sub, rows_per_sub)
        pltpu.sync_copy(acc_sh.at[rs], out_hbm.at[cid, rs])

    return agg_kernel


def _dinv_from_hist(hist_ref, n):
    deg = hist_ref[0, 0:n, 0:1] + hist_ref[1, 0:n, 0:1] + 1.0
    return lax.rsqrt(deg)


def _mm(a, b):
    return jnp.dot(a, b, preferred_element_type=jnp.float32)


def _tc_mm1(x_ref, w_ref, o_ref):
    o_ref[...] = _mm(x_ref[...], w_ref[...])


def _tc_scale(mm_ref, hist_ref, o_ref):
    o_ref[...] = _dinv_from_hist(hist_ref, mm_ref.shape[0]) * mm_ref[...]


def _agg_sum(agg_ref, n):
    return agg_ref[0, 0:n, :] + agg_ref[1, 0:n, :]


def _tc_layer(agg_ref, hs_ref, hist_ref, b_ref, w_ref, x_ref, hn_ref):
    n = hs_ref.shape[0]
    dinv = _dinv_from_hist(hist_ref, n)
    xl = _elu(dinv * (_agg_sum(agg_ref, n) + hs_ref[...]) + b_ref[...])
    x_ref[...] = xl
    hn_ref[...] = dinv * _mm(xl, w_ref[...])


def _tc_final(x1_ref, x2_ref, agg_ref, hs_ref, hist_ref, b3_ref, batch_ref,
              p1_ref, pb1_ref, g1_ref, bt1_ref, p2_ref, pb2_ref, g2_ref,
              bt2_ref, m1_ref, mb1_ref, g3_ref, bt3_ref, m2_ref, mb2_ref,
              g4_ref, bt4_ref, m3_ref, mb3_ref, o_ref):
    n = hs_ref.shape[0]
    dinv = _dinv_from_hist(hist_ref, n)
    x3 = _elu(dinv * (_agg_sum(agg_ref, n) + hs_ref[...]) + b3_ref[...])

    h = x1_ref.shape[1]
    z1 = _lrelu(_mm(x1_ref[...], p1_ref[0:h]) + _mm(x2_ref[...], p1_ref[h:2 * h])
                + _mm(x3, p1_ref[2 * h:3 * h]) + pb1_ref[...])
    nn = jnp.float32(z1.shape[0])
    mu1 = jnp.sum(z1, axis=0, keepdims=True) / nn
    var1 = jnp.sum(z1 * z1, axis=0, keepdims=True) / nn - mu1 * mu1
    a1 = lax.rsqrt(var1 + 1e-5) * g1_ref[...]
    c1 = bt1_ref[...] - mu1 * a1

    z2 = _lrelu(_mm(z1 * a1 + c1, p2_ref[...]) + pb2_ref[...])
    mu2 = jnp.sum(z2, axis=0, keepdims=True) / nn
    var2 = jnp.sum(z2 * z2, axis=0, keepdims=True) / nn - mu2 * mu2
    a2 = lax.rsqrt(var2 + 1e-5) * g2_ref[...]
    c2 = bt2_ref[...] - mu2 * a2

    g = o_ref.shape[0]
    seg_ids = lax.broadcasted_iota(jnp.int32, (g, 1), 0)
    onehot = (seg_ids == batch_ref[...]).astype(jnp.float32)
    seg = _mm(onehot, z2)
    cnt = jnp.sum(onehot, axis=1, keepdims=True)
    cntm = jnp.maximum(cnt, 1.0)
    pooled = (seg * a2 + c2 * cnt) / cntm

    def bn_small(v, gg, bb):
        gn = jnp.float32(v.shape[0])
        mu = jnp.sum(v, axis=0, keepdims=True) / gn
        var = jnp.sum(v * v, axis=0, keepdims=True) / gn - mu * mu
        return (v - mu) * lax.rsqrt(var + 1e-5) * gg + bb

    m = bn_small(_lrelu(_mm(pooled, m1_ref[...]) + mb1_ref[...]),
                 g3_ref[...], bt3_ref[...])
    m = bn_small(_lrelu(_mm(m, m2_ref[...]) + mb2_ref[...]),
                 g4_ref[...], bt4_ref[...])
    m = _mm(m, m3_ref[...]) + mb3_ref[...]
    mx = jnp.max(m, axis=1, keepdims=True)
    s = m - mx
    o_ref[...] = s - jnp.log(jnp.sum(jnp.exp(s), axis=1, keepdims=True))


def _call(body, out_shape, *args):
    return pl.pallas_call(body, out_shape=out_shape)(*args)


def kernel(x, edge_index, batch, W1, b1, W2, b2, W3, b3, P1, pb1, g1, bt1,
           P2, pb2, g2, bt2, M1, mb1, g3, bt3, M2, mb2, g4, bt4, M3, mb3):
    n, d = x.shape
    e = edge_index.shape[1]
    h = W1.shape[1]
    G = 64
    O = M3.shape[1]
    rows_per_sub = _pad_n(n) // NS

    ei = edge_index.astype(jnp.int32)
    epad = _pad_e(e)
    npad = _pad_n(n)
    assert npad > n
    pad = epad - e
    src_p = jnp.concatenate([ei[0], jnp.zeros((pad,), jnp.int32)])
    trash = n + (jnp.arange(pad, dtype=jnp.int32) % (npad - n))
    dst_p = jnp.concatenate([ei[1], trash])
    src2d = src_p.reshape(epad // CH, CH)
    dst2d = dst_p.reshape(epad // CH, CH)
    zeros_h = jnp.zeros((rows_per_sub, h), jnp.float32)
    batch2d = batch.astype(jnp.int32).reshape(1, n)

    f32 = jnp.float32
    nd = jax.ShapeDtypeStruct((n, h), f32)

    hist = _make_sc_degree(n, epad)(zeros_h, dst2d)
    mm1 = _call(_tc_mm1, nd, x, W1)
    h1s = _call(_tc_scale, nd, mm1, hist)

    agg1 = _make_sc_agg(n, epad, h)(zeros_h, h1s, src2d, dst2d)
    x1, h2s = _call(_tc_layer, (nd, nd), agg1, h1s, hist,
                    b1.reshape(1, h), W2)
    agg2 = _make_sc_agg(n, epad, h)(zeros_h, h2s, src2d, dst2d)
    x2, h3s = _call(_tc_layer, (nd, nd), agg2, h2s, hist,
                    b2.reshape(1, h), W3)
    agg3 = _make_sc_agg(n, epad, h)(zeros_h, h3s, src2d, dst2d)

    out = _call(_tc_final, jax.ShapeDtypeStruct((G, O), f32),
                x1, x2, agg3, h3s, hist, b3.reshape(1, h), batch2d,
                P1, pb1.reshape(1, h), g1.reshape(1, h), bt1.reshape(1, h),
                P2, pb2.reshape(1, h), g2.reshape(1, h), bt2.reshape(1, h),
                M1, mb1.reshape(1, h), g3.reshape(1, h), bt3.reshape(1, h),
                M2, mb2.reshape(1, h), g4.reshape(1, h), bt4.reshape(1, h),
                M3, mb3.reshape(1, O))
    return out


# speedup vs baseline: 1.0001x; 1.0001x over previous
import functools

import jax
import jax.numpy as jnp
from jax import lax
from jax.experimental import pallas as pl
from jax.experimental.pallas import tpu as pltpu
from jax.experimental.pallas import tpu_sc as plsc

NC = 2
NS = 16
CH = 128


def _pad_n(n):
    q = NS * 8
    return ((n + q) // q) * q


def _pad_e(e):
    q = NC * NS * CH * 8
    return ((e + q - 1) // q) * q


def _elu(v):
    return jnp.where(v > 0, v, jnp.exp(jnp.minimum(v, 0.0)) - 1.0)


def _lrelu(v):
    return jnp.where(v >= 0, v, 0.01 * v)


@functools.lru_cache(maxsize=None)
def _make_sc_degree(n, e):
    per_sub = e // (NC * NS)
    n_chunks = per_sub // CH
    npad = _pad_n(n)
    assert per_sub % CH == 0 and n_chunks % 8 == 0
    rows_per_sub = npad // NS
    mesh = plsc.VectorSubcoreMesh(core_axis_name="c", subcore_axis_name="s")

    @functools.partial(
        pl.kernel,
        mesh=mesh,
        out_type=jax.ShapeDtypeStruct((NC, npad, 128), jnp.float32),
        scratch_types=[
            pltpu.VMEM((n_chunks, CH), jnp.int32),
            pltpu.VMEM((CH, 128), jnp.float32),
            pltpu.VMEM_SHARED((npad, 128), jnp.float32),
        ],
    )
    def deg_kernel(zeros_hbm, dst_hbm, out_hbm, didx, ones_v, acc_sh):
        cid = lax.axis_index("c")
        sid = lax.axis_index("s")
        wrow = (cid * NS + sid) * n_chunks
        pltpu.sync_copy(dst_hbm.at[pl.ds(wrow, n_chunks)], didx)

        @pl.loop(0, CH)
        def _(i):
            @pl.loop(0, 128, step=16)
            def _(j):
                ones_v[i, pl.ds(j, 16)] = jnp.ones((16,), jnp.float32)

        pltpu.sync_copy(zeros_hbm, acc_sh.at[pl.ds(sid * rows_per_sub, rows_per_sub)])
        plsc.subcore_barrier()

        @pl.loop(0, n_chunks)
        def _(k):
            pltpu.sync_copy(ones_v, acc_sh.at[didx.at[k]], add=True)

        plsc.subcore_barrier()
        rs = pl.ds(sid * rows_per_sub, rows_per_sub)
        pltpu.sync_copy(acc_sh.at[rs], out_hbm.at[cid, rs])

    return deg_kernel


@functools.lru_cache(maxsize=None)
def _make_sc_agg(n, e, d):
    per_sub = e // (NC * NS)
    n_chunks = per_sub // CH
    npad = _pad_n(n)
    nblk = n_chunks // 8
    assert per_sub % CH == 0 and n_chunks % 16 == 0 and nblk >= 2
    rows_per_sub = npad // NS
    mesh = plsc.VectorSubcoreMesh(core_axis_name="c", subcore_axis_name="s")

    @functools.partial(
        pl.kernel,
        mesh=mesh,
        out_type=jax.ShapeDtypeStruct((NC, npad, d), jnp.float32),
        scratch_types=[
            pltpu.VMEM((n_chunks, CH), jnp.int32),
            pltpu.VMEM((8, CH), jnp.int32),
            pltpu.VMEM((8, CH), jnp.int32),
            pltpu.VMEM((CH, d), jnp.float32),
            pltpu.VMEM((CH, d), jnp.float32),
            pltpu.SemaphoreType.DMA,
            pltpu.SemaphoreType.DMA,
            pltpu.SemaphoreType.DMA,
            pltpu.VMEM_SHARED((npad, d), jnp.float32),
        ],
    )
    def agg_kernel(zeros_hbm, h_hbm, src_hbm, dst_hbm, out_hbm,
                   didx, sblk0, sblk1, buf0, buf1, g0, g1, isem, acc_sh):
        sblks = (sblk0, sblk1)
        bufs = (buf0, buf1)
        gsems = (g0, g1)
        cid = lax.axis_index("c")
        sid = lax.axis_index("s")
        wrow = (cid * NS + sid) * n_chunks
        pltpu.sync_copy(dst_hbm.at[pl.ds(wrow, n_chunks)], didx)
        pltpu.sync_copy(src_hbm.at[pl.ds(wrow, 8)], sblk0)
        pltpu.sync_copy(src_hbm.at[pl.ds(wrow + 8, 8)], sblk1)
        pltpu.sync_copy(zeros_hbm, acc_sh.at[pl.ds(sid * rows_per_sub, rows_per_sub)])
        plsc.subcore_barrier()

        def fire(sb, r, b):
            pltpu.async_copy(h_hbm.at[sblks[sb].at[r]], bufs[b], gsems[b])

        def wait_g(sb, r, b):
            pltpu.make_async_copy(h_hbm.at[sblks[sb].at[r]], bufs[b], gsems[b]).wait()

        def idx_cp(j, sb):
            return pltpu.make_async_copy(
                src_hbm.at[pl.ds(wrow + j * 8, 8)], sblks[sb], isem)

        @pl.loop(0, nblk, step=2)
        def _(j):
            c0 = j * 8

            @pl.when(j > 0)
            def _():
                pltpu.sync_copy(src_hbm.at[pl.ds(wrow + j * 8, 8)], sblk0)
                pltpu.sync_copy(src_hbm.at[pl.ds(wrow + (j + 1) * 8, 8)], sblk1)

            fire(0, 0, 0)
            fire(0, 1, 1)
            for t in range(16):
                sb, r = divmod(t, 8)
                b = t % 2
                wait_g(sb, r, b)
                pltpu.sync_copy(bufs[b], acc_sh.at[didx.at[c0 + t]], add=True)
                if t < 14:
                    sb2, r2 = divmod(t + 2, 8)
                    fire(sb2, r2, b)

        plsc.subcore_barrier()
        rs = pl.ds(sid * rows_per_sub, rows_per_sub)
        pltpu.sync_copy(acc_sh.at[rs], out_hbm.at[cid, rs])

    return agg_kernel


def _dinv_from_hist(hist_ref, n):
    deg = hist_ref[0, 0:n, 0:1] + hist_ref[1, 0:n, 0:1] + 1.0
    return lax.rsqrt(deg)


def _mm(a, b):
    return jnp.dot(a, b, preferred_element_type=jnp.float32)


def _tc_mm1(x_ref, w_ref, o_ref):
    o_ref[...] = _mm(x_ref[...], w_ref[...])


def _tc_scale(mm_ref, hist_ref, o_ref):
    o_ref[...] = _dinv_from_hist(hist_ref, mm_ref.shape[0]) * mm_ref[...]


def _agg_sum(agg_ref, n):
    return agg_ref[0, 0:n, :] + agg_ref[1, 0:n, :]


def _tc_layer(agg_ref, hs_ref, hist_ref, b_ref, w_ref, x_ref, hn_ref):
    n = hs_ref.shape[0]
    dinv = _dinv_from_hist(hist_ref, n)
    xl = _elu(dinv * (_agg_sum(agg_ref, n) + hs_ref[...]) + b_ref[...])
    x_ref[...] = xl
    hn_ref[...] = dinv * _mm(xl, w_ref[...])


def _tc_final(x1_ref, x2_ref, agg_ref, hs_ref, hist_ref, b3_ref, batch_ref,
              p1_ref, pb1_ref, g1_ref, bt1_ref, p2_ref, pb2_ref, g2_ref,
              bt2_ref, m1_ref, mb1_ref, g3_ref, bt3_ref, m2_ref, mb2_ref,
              g4_ref, bt4_ref, m3_ref, mb3_ref, o_ref):
    n = hs_ref.shape[0]
    dinv = _dinv_from_hist(hist_ref, n)
    x3 = _elu(dinv * (_agg_sum(agg_ref, n) + hs_ref[...]) + b3_ref[...])

    h = x1_ref.shape[1]
    z1 = _lrelu(_mm(x1_ref[...], p1_ref[0:h]) + _mm(x2_ref[...], p1_ref[h:2 * h])
                + _mm(x3, p1_ref[2 * h:3 * h]) + pb1_ref[...])
    nn = jnp.float32(z1.shape[0])
    mu1 = jnp.sum(z1, axis=0, keepdims=True) / nn
    var1 = jnp.sum(z1 * z1, axis=0, keepdims=True) / nn - mu1 * mu1
    a1 = lax.rsqrt(var1 + 1e-5) * g1_ref[...]
    c1 = bt1_ref[...] - mu1 * a1

    z2 = _lrelu(_mm(z1 * a1 + c1, p2_ref[...]) + pb2_ref[...])
    mu2 = jnp.sum(z2, axis=0, keepdims=True) / nn
    var2 = jnp.sum(z2 * z2, axis=0, keepdims=True) / nn - mu2 * mu2
    a2 = lax.rsqrt(var2 + 1e-5) * g2_ref[...]
    c2 = bt2_ref[...] - mu2 * a2

    g = o_ref.shape[0]
    seg_ids = lax.broadcasted_iota(jnp.int32, (g, 1), 0)
    onehot = (seg_ids == batch_ref[...]).astype(jnp.float32)
    seg = _mm(onehot, z2)
    cnt = jnp.sum(onehot, axis=1, keepdims=True)
    cntm = jnp.maximum(cnt, 1.0)
    pooled = (seg * a2 + c2 * cnt) / cntm

    def bn_small(v, gg, bb):
        gn = jnp.float32(v.shape[0])
        mu = jnp.sum(v, axis=0, keepdims=True) / gn
        var = jnp.sum(v * v, axis=0, keepdims=True) / gn - mu * mu
        return (v - mu) * lax.rsqrt(var + 1e-5) * gg + bb

    m = bn_small(_lrelu(_mm(pooled, m1_ref[...]) + mb1_ref[...]),
                 g3_ref[...], bt3_ref[...])
    m = bn_small(_lrelu(_mm(m, m2_ref[...]) + mb2_ref[...]),
                 g4_ref[...], bt4_ref[...])
    m = _mm(m, m3_ref[...]) + mb3_ref[...]
    mx = jnp.max(m, axis=1, keepdims=True)
    s = m - mx
    o_ref[...] = s - jnp.log(jnp.sum(jnp.exp(s), axis=1, keepdims=True))


def _call(body, out_shape, *args):
    return pl.pallas_call(body, out_shape=out_shape)(*args)


def kernel(x, edge_index, batch, W1, b1, W2, b2, W3, b3, P1, pb1, g1, bt1,
           P2, pb2, g2, bt2, M1, mb1, g3, bt3, M2, mb2, g4, bt4, M3, mb3):
    n, d = x.shape
    e = edge_index.shape[1]
    h = W1.shape[1]
    G = 64
    O = M3.shape[1]
    rows_per_sub = _pad_n(n) // NS

    ei = edge_index.astype(jnp.int32)
    epad = _pad_e(e)
    npad = _pad_n(n)
    assert npad > n
    pad = epad - e
    src_p = jnp.concatenate([ei[0], jnp.zeros((pad,), jnp.int32)])
    trash = n + (jnp.arange(pad, dtype=jnp.int32) % (npad - n))
    dst_p = jnp.concatenate([ei[1], trash])
    src2d = src_p.reshape(epad // CH, CH)
    dst2d = dst_p.reshape(epad // CH, CH)
    zeros_h = jnp.zeros((rows_per_sub, h), jnp.float32)
    batch2d = batch.astype(jnp.int32).reshape(1, n)

    f32 = jnp.float32
    nd = jax.ShapeDtypeStruct((n, h), f32)

    hist = _make_sc_degree(n, epad)(zeros_h, dst2d)
    mm1 = _call(_tc_mm1, nd, x, W1)
    h1s = _call(_tc_scale, nd, mm1, hist)

    agg1 = _make_sc_agg(n, epad, h)(zeros_h, h1s, src2d, dst2d)
    x1, h2s = _call(_tc_layer, (nd, nd), agg1, h1s, hist,
                    b1.reshape(1, h), W2)
    agg2 = _make_sc_agg(n, epad, h)(zeros_h, h2s, src2d, dst2d)
    x2, h3s = _call(_tc_layer, (nd, nd), agg2, h2s, hist,
                    b2.reshape(1, h), W3)
    agg3 = _make_sc_agg(n, epad, h)(zeros_h, h3s, src2d, dst2d)

    out = _call(_tc_final, jax.ShapeDtypeStruct((G, O), f32),
                x1, x2, agg3, h3s, hist, b3.reshape(1, h), batch2d,
                P1, pb1.reshape(1, h), g1.reshape(1, h), bt1.reshape(1, h),
                P2, pb2.reshape(1, h), g2.reshape(1, h), bt2.reshape(1, h),
                M1, mb1.reshape(1, h), g3.reshape(1, h), bt3.reshape(1, h),
                M2, mb2.reshape(1, h), g4.reshape(1, h), bt4.reshape(1, h),
                M3, mb3.reshape(1, O))
    return out


# speedup vs baseline: 2.9125x; 2.9124x over previous
import functools

import jax
import jax.numpy as jnp
from jax import lax
from jax.experimental import pallas as pl
from jax.experimental.pallas import tpu as pltpu
from jax.experimental.pallas import tpu_sc as plsc

NC = 2
NS = 16
CH = 128


def _pad_n(n):
    q = NS * 8
    return ((n + q) // q) * q


def _pad_e(e):
    q = NC * NS * CH * 8
    return ((e + q - 1) // q) * q


def _elu(v):
    return jnp.where(v > 0, v, jnp.exp(jnp.minimum(v, 0.0)) - 1.0)


def _lrelu(v):
    return jnp.where(v >= 0, v, 0.01 * v)


@functools.lru_cache(maxsize=None)
def _make_sc_degree(n, e):
    per_sub = e // (NC * NS)
    n_chunks = per_sub // CH
    npad = _pad_n(n)
    assert per_sub % CH == 0 and n_chunks % 8 == 0
    rows_per_sub = npad // NS
    mesh = plsc.VectorSubcoreMesh(core_axis_name="c", subcore_axis_name="s")

    @functools.partial(
        pl.kernel,
        mesh=mesh,
        out_type=jax.ShapeDtypeStruct((NC, npad, 128), jnp.float32),
        scratch_types=[
            pltpu.VMEM((n_chunks, CH), jnp.int32),
            pltpu.VMEM((CH, 128), jnp.float32),
            pltpu.VMEM_SHARED((npad, 128), jnp.float32),
        ],
    )
    def deg_kernel(zeros_hbm, dst_hbm, out_hbm, didx, ones_v, acc_sh):
        cid = lax.axis_index("c")
        sid = lax.axis_index("s")
        wrow = (cid * NS + sid) * n_chunks
        pltpu.sync_copy(dst_hbm.at[pl.ds(wrow, n_chunks)], didx)

        @pl.loop(0, CH)
        def _(i):
            @pl.loop(0, 128, step=16)
            def _(j):
                ones_v[i, pl.ds(j, 16)] = jnp.ones((16,), jnp.float32)

        pltpu.sync_copy(zeros_hbm, acc_sh.at[pl.ds(sid * rows_per_sub, rows_per_sub)])
        plsc.subcore_barrier()

        @pl.loop(0, n_chunks)
        def _(k):
            pltpu.sync_copy(ones_v, acc_sh.at[didx.at[k]], add=True)

        plsc.subcore_barrier()
        rs = pl.ds(sid * rows_per_sub, rows_per_sub)
        pltpu.sync_copy(acc_sh.at[rs], out_hbm.at[cid, rs])

    return deg_kernel


@functools.lru_cache(maxsize=None)
def _make_sc_agg(n, e, d):
    CA = 80
    per_sub = e // (NC * NS)
    n_chunks = per_sub // CA
    npad = _pad_n(n)
    assert per_sub % CA == 0 and n_chunks % 2 == 1 and per_sub % 8 == 0
    rows_per_sub = npad // NS
    mesh = plsc.VectorSubcoreMesh(core_axis_name="c", subcore_axis_name="s")

    @functools.partial(
        pl.kernel,
        mesh=mesh,
        out_type=jax.ShapeDtypeStruct((NC, npad, d), jnp.float32),
        scratch_types=[
            pltpu.VMEM((CA,), jnp.int32),
            pltpu.VMEM((CA,), jnp.int32),
            pltpu.VMEM((CA,), jnp.int32),
            pltpu.VMEM((CA,), jnp.int32),
            pltpu.VMEM((CA, d), jnp.float32),
            pltpu.VMEM((CA, d), jnp.float32),
            pltpu.SemaphoreType.DMA,
            pltpu.SemaphoreType.DMA,
            pltpu.SemaphoreType.DMA,
            pltpu.SemaphoreType.DMA,
            pltpu.SemaphoreType.DMA,
            pltpu.SemaphoreType.DMA,
            pltpu.VMEM_SHARED((npad, d), jnp.float32),
        ],
    )
    def agg_kernel(zeros_hbm, h_hbm, src_hbm, dst_hbm, out_hbm,
                   src0, dst0, src1, dst1, buf0, buf1,
                   ss0, ds0, ss1, ds1, g0, g1, acc_sh):
        srcs = (src0, src1)
        dsts = (dst0, dst1)
        bufs = (buf0, buf1)
        ssems = (ss0, ss1)
        dsems = (ds0, ds1)
        gsems = (g0, g1)
        cid = lax.axis_index("c")
        sid = lax.axis_index("s")
        base = (cid * NS + sid) * per_sub
        pltpu.sync_copy(zeros_hbm, acc_sh.at[pl.ds(sid * rows_per_sub, rows_per_sub)])

        def src_cp(k, p):
            return pltpu.make_async_copy(
                src_hbm.at[pl.ds(base + k * CA, CA)], srcs[p], ssems[p])

        def dst_cp(k, p):
            return pltpu.make_async_copy(
                dst_hbm.at[pl.ds(base + k * CA, CA)], dsts[p], dsems[p])

        def gat_cp(p):
            return pltpu.make_async_copy(h_hbm.at[srcs[p]], bufs[p], gsems[p])

        src_cp(0, 0).start()
        dst_cp(0, 0).start()
        src_cp(1, 1).start()
        dst_cp(1, 1).start()
        plsc.subcore_barrier()
        src_cp(0, 0).wait()
        gat_cp(0).start()
        src_cp(1, 1).wait()
        gat_cp(1).start()

        def half(k, p):
            gat_cp(p).wait()
            last = k + 2 >= n_chunks

            @pl.when(jnp.logical_not(last))
            def _():
                src_cp(k + 2, p).start()
            dst_cp(k, p).wait()
            pltpu.sync_copy(bufs[p], acc_sh.at[dsts[p]], add=True)

            @pl.when(jnp.logical_not(last))
            def _():
                dst_cp(k + 2, p).start()
                src_cp(k + 2, p).wait()
                gat_cp(p).start()

        @pl.loop(0, n_chunks - 1, step=2)
        def _(k):
            half(k, 0)
            half(k + 1, 1)

        half(n_chunks - 1, 0)

        plsc.subcore_barrier()
        rs = pl.ds(sid * rows_per_sub, rows_per_sub)
        pltpu.sync_copy(acc_sh.at[rs], out_hbm.at[cid, rs])

    return agg_kernel


def _dinv_from_hist(hist_ref, n):
    deg = hist_ref[0, 0:n, 0:1] + hist_ref[1, 0:n, 0:1] + 1.0
    return lax.rsqrt(deg)


def _mm(a, b):
    return jnp.dot(a, b, preferred_element_type=jnp.float32)


def _tc_mm1(x_ref, w_ref, o_ref):
    o_ref[...] = _mm(x_ref[...], w_ref[...])


def _tc_scale(mm_ref, hist_ref, o_ref):
    o_ref[...] = _dinv_from_hist(hist_ref, mm_ref.shape[0]) * mm_ref[...]


def _agg_sum(agg_ref, n):
    return agg_ref[0, 0:n, :] + agg_ref[1, 0:n, :]


def _tc_layer(agg_ref, hs_ref, hist_ref, b_ref, w_ref, x_ref, hn_ref):
    n = hs_ref.shape[0]
    dinv = _dinv_from_hist(hist_ref, n)
    xl = _elu(dinv * (_agg_sum(agg_ref, n) + hs_ref[...]) + b_ref[...])
    x_ref[...] = xl
    hn_ref[...] = dinv * _mm(xl, w_ref[...])


def _tc_final(x1_ref, x2_ref, agg_ref, hs_ref, hist_ref, b3_ref, batch_ref,
              p1_ref, pb1_ref, g1_ref, bt1_ref, p2_ref, pb2_ref, g2_ref,
              bt2_ref, m1_ref, mb1_ref, g3_ref, bt3_ref, m2_ref, mb2_ref,
              g4_ref, bt4_ref, m3_ref, mb3_ref, o_ref):
    n = hs_ref.shape[0]
    dinv = _dinv_from_hist(hist_ref, n)
    x3 = _elu(dinv * (_agg_sum(agg_ref, n) + hs_ref[...]) + b3_ref[...])

    h = x1_ref.shape[1]
    z1 = _lrelu(_mm(x1_ref[...], p1_ref[0:h]) + _mm(x2_ref[...], p1_ref[h:2 * h])
                + _mm(x3, p1_ref[2 * h:3 * h]) + pb1_ref[...])
    nn = jnp.float32(z1.shape[0])
    mu1 = jnp.sum(z1, axis=0, keepdims=True) / nn
    var1 = jnp.sum(z1 * z1, axis=0, keepdims=True) / nn - mu1 * mu1
    a1 = lax.rsqrt(var1 + 1e-5) * g1_ref[...]
    c1 = bt1_ref[...] - mu1 * a1

    z2 = _lrelu(_mm(z1 * a1 + c1, p2_ref[...]) + pb2_ref[...])
    mu2 = jnp.sum(z2, axis=0, keepdims=True) / nn
    var2 = jnp.sum(z2 * z2, axis=0, keepdims=True) / nn - mu2 * mu2
    a2 = lax.rsqrt(var2 + 1e-5) * g2_ref[...]
    c2 = bt2_ref[...] - mu2 * a2

    g = o_ref.shape[0]
    seg_ids = lax.broadcasted_iota(jnp.int32, (g, 1), 0)
    onehot = (seg_ids == batch_ref[...]).astype(jnp.float32)
    seg = _mm(onehot, z2)
    cnt = jnp.sum(onehot, axis=1, keepdims=True)
    cntm = jnp.maximum(cnt, 1.0)
    pooled = (seg * a2 + c2 * cnt) / cntm

    def bn_small(v, gg, bb):
        gn = jnp.float32(v.shape[0])
        mu = jnp.sum(v, axis=0, keepdims=True) / gn
        var = jnp.sum(v * v, axis=0, keepdims=True) / gn - mu * mu
        return (v - mu) * lax.rsqrt(var + 1e-5) * gg + bb

    m = bn_small(_lrelu(_mm(pooled, m1_ref[...]) + mb1_ref[...]),
                 g3_ref[...], bt3_ref[...])
    m = bn_small(_lrelu(_mm(m, m2_ref[...]) + mb2_ref[...]),
                 g4_ref[...], bt4_ref[...])
    m = _mm(m, m3_ref[...]) + mb3_ref[...]
    mx = jnp.max(m, axis=1, keepdims=True)
    s = m - mx
    o_ref[...] = s - jnp.log(jnp.sum(jnp.exp(s), axis=1, keepdims=True))


def _call(body, out_shape, *args):
    return pl.pallas_call(body, out_shape=out_shape)(*args)


def kernel(x, edge_index, batch, W1, b1, W2, b2, W3, b3, P1, pb1, g1, bt1,
           P2, pb2, g2, bt2, M1, mb1, g3, bt3, M2, mb2, g4, bt4, M3, mb3):
    n, d = x.shape
    e = edge_index.shape[1]
    h = W1.shape[1]
    G = 64
    O = M3.shape[1]
    rows_per_sub = _pad_n(n) // NS

    ei = edge_index.astype(jnp.int32)
    epad = _pad_e(e)
    npad = _pad_n(n)
    assert npad > n
    pad = epad - e
    src1d = ei[0]
    dst1d = ei[1]
    trash = n + (jnp.arange(pad, dtype=jnp.int32) % (npad - n))
    dst2d = jnp.concatenate([ei[1], trash]).reshape(epad // CH, CH)
    zeros_h = jnp.zeros((rows_per_sub, h), jnp.float32)
    batch2d = batch.astype(jnp.int32).reshape(1, n)

    f32 = jnp.float32
    nd = jax.ShapeDtypeStruct((n, h), f32)

    hist = _make_sc_degree(n, epad)(zeros_h, dst2d)
    mm1 = _call(_tc_mm1, nd, x, W1)
    h1s = _call(_tc_scale, nd, mm1, hist)

    agg1 = _make_sc_agg(n, e, h)(zeros_h, h1s, src1d, dst1d)
    x1, h2s = _call(_tc_layer, (nd, nd), agg1, h1s, hist,
                    b1.reshape(1, h), W2)
    agg2 = _make_sc_agg(n, e, h)(zeros_h, h2s, src1d, dst1d)
    x2, h3s = _call(_tc_layer, (nd, nd), agg2, h2s, hist,
                    b2.reshape(1, h), W3)
    agg3 = _make_sc_agg(n, e, h)(zeros_h, h3s, src1d, dst1d)

    out = _call(_tc_final, jax.ShapeDtypeStruct((G, O), f32),
                x1, x2, agg3, h3s, hist, b3.reshape(1, h), batch2d,
                P1, pb1.reshape(1, h), g1.reshape(1, h), bt1.reshape(1, h),
                P2, pb2.reshape(1, h), g2.reshape(1, h), bt2.reshape(1, h),
                M1, mb1.reshape(1, h), g3.reshape(1, h), bt3.reshape(1, h),
                M2, mb2.reshape(1, h), g4.reshape(1, h), bt4.reshape(1, h),
                M3, mb3.reshape(1, O))
    return out


# speedup vs baseline: 2.9317x; 1.0066x over previous
import functools

import jax
import jax.numpy as jnp
from jax import lax
from jax.experimental import pallas as pl
from jax.experimental.pallas import tpu as pltpu
from jax.experimental.pallas import tpu_sc as plsc

NC = 2
NS = 16
def _pad_n(n):
    q = NS * 8
    return ((n + q) // q) * q


def _elu(v):
    return jnp.where(v > 0, v, jnp.exp(jnp.minimum(v, 0.0)) - 1.0)


def _lrelu(v):
    return jnp.where(v >= 0, v, 0.01 * v)


@functools.lru_cache(maxsize=None)
def _make_sc_degree(n, e):
    CA = 80
    per_sub = e // (NC * NS)
    n_chunks = per_sub // CA
    npad = _pad_n(n)
    assert per_sub % CA == 0 and n_chunks % 2 == 1 and per_sub % 8 == 0
    rows_per_sub = npad // NS
    mesh = plsc.VectorSubcoreMesh(core_axis_name="c", subcore_axis_name="s")

    @functools.partial(
        pl.kernel,
        mesh=mesh,
        out_type=jax.ShapeDtypeStruct((NC, npad, 128), jnp.float32),
        scratch_types=[
            pltpu.VMEM((CA,), jnp.int32),
            pltpu.VMEM((CA,), jnp.int32),
            pltpu.VMEM((CA, 128), jnp.float32),
            pltpu.SemaphoreType.DMA,
            pltpu.SemaphoreType.DMA,
            pltpu.VMEM_SHARED((npad, 128), jnp.float32),
        ],
    )
    def deg_kernel(zeros_hbm, dst_hbm, out_hbm, dst0, dst1, ones_v,
                   ds0, ds1, acc_sh):
        dsts = (dst0, dst1)
        dsems = (ds0, ds1)
        cid = lax.axis_index("c")
        sid = lax.axis_index("s")
        base = (cid * NS + sid) * per_sub

        def dst_cp(k, p):
            return pltpu.make_async_copy(
                dst_hbm.at[pl.ds(base + k * CA, CA)], dsts[p], dsems[p])

        dst_cp(0, 0).start()
        dst_cp(1, 1).start()

        @pl.loop(0, CA)
        def _(i):
            @pl.loop(0, 128, step=16)
            def _(j):
                ones_v[i, pl.ds(j, 16)] = jnp.ones((16,), jnp.float32)

        pltpu.sync_copy(zeros_hbm, acc_sh.at[pl.ds(sid * rows_per_sub, rows_per_sub)])
        plsc.subcore_barrier()

        def half(k, p):
            dst_cp(k, p).wait()
            pltpu.sync_copy(ones_v, acc_sh.at[dsts[p]], add=True)

            @pl.when(jnp.logical_not(k + 2 >= n_chunks))
            def _():
                dst_cp(k + 2, p).start()

        @pl.loop(0, n_chunks - 1, step=2)
        def _(k):
            half(k, 0)
            half(k + 1, 1)

        half(n_chunks - 1, 0)

        plsc.subcore_barrier()
        rs = pl.ds(sid * rows_per_sub, rows_per_sub)
        pltpu.sync_copy(acc_sh.at[rs], out_hbm.at[cid, rs])

    return deg_kernel


@functools.lru_cache(maxsize=None)
def _make_sc_agg(n, e, d):
    CA = 80
    per_sub = e // (NC * NS)
    n_chunks = per_sub // CA
    npad = _pad_n(n)
    assert per_sub % CA == 0 and n_chunks % 2 == 1 and per_sub % 8 == 0
    rows_per_sub = npad // NS
    mesh = plsc.VectorSubcoreMesh(core_axis_name="c", subcore_axis_name="s")

    @functools.partial(
        pl.kernel,
        mesh=mesh,
        out_type=jax.ShapeDtypeStruct((NC, npad, d), jnp.float32),
        scratch_types=[
            pltpu.VMEM((CA,), jnp.int32),
            pltpu.VMEM((CA,), jnp.int32),
            pltpu.VMEM((CA,), jnp.int32),
            pltpu.VMEM((CA,), jnp.int32),
            pltpu.VMEM((CA, d), jnp.float32),
            pltpu.VMEM((CA, d), jnp.float32),
            pltpu.SemaphoreType.DMA,
            pltpu.SemaphoreType.DMA,
            pltpu.SemaphoreType.DMA,
            pltpu.SemaphoreType.DMA,
            pltpu.SemaphoreType.DMA,
            pltpu.SemaphoreType.DMA,
            pltpu.VMEM_SHARED((npad, d), jnp.float32),
        ],
    )
    def agg_kernel(zeros_hbm, h_hbm, src_hbm, dst_hbm, out_hbm,
                   src0, dst0, src1, dst1, buf0, buf1,
                   ss0, ds0, ss1, ds1, g0, g1, acc_sh):
        srcs = (src0, src1)
        dsts = (dst0, dst1)
        bufs = (buf0, buf1)
        ssems = (ss0, ss1)
        dsems = (ds0, ds1)
        gsems = (g0, g1)
        cid = lax.axis_index("c")
        sid = lax.axis_index("s")
        base = (cid * NS + sid) * per_sub
        pltpu.sync_copy(zeros_hbm, acc_sh.at[pl.ds(sid * rows_per_sub, rows_per_sub)])

        def src_cp(k, p):
            return pltpu.make_async_copy(
                src_hbm.at[pl.ds(base + k * CA, CA)], srcs[p], ssems[p])

        def dst_cp(k, p):
            return pltpu.make_async_copy(
                dst_hbm.at[pl.ds(base + k * CA, CA)], dsts[p], dsems[p])

        def gat_cp(p):
            return pltpu.make_async_copy(h_hbm.at[srcs[p]], bufs[p], gsems[p])

        src_cp(0, 0).start()
        dst_cp(0, 0).start()
        src_cp(1, 1).start()
        dst_cp(1, 1).start()
        plsc.subcore_barrier()
        src_cp(0, 0).wait()
        gat_cp(0).start()
        src_cp(1, 1).wait()
        gat_cp(1).start()

        def half(k, p):
            gat_cp(p).wait()
            last = k + 2 >= n_chunks

            @pl.when(jnp.logical_not(last))
            def _():
                src_cp(k + 2, p).start()
            dst_cp(k, p).wait()
            pltpu.sync_copy(bufs[p], acc_sh.at[dsts[p]], add=True)

            @pl.when(jnp.logical_not(last))
            def _():
                dst_cp(k + 2, p).start()
                src_cp(k + 2, p).wait()
                gat_cp(p).start()

        @pl.loop(0, n_chunks - 1, step=2)
        def _(k):
            half(k, 0)
            half(k + 1, 1)

        half(n_chunks - 1, 0)

        plsc.subcore_barrier()
        rs = pl.ds(sid * rows_per_sub, rows_per_sub)
        pltpu.sync_copy(acc_sh.at[rs], out_hbm.at[cid, rs])

    return agg_kernel


def _dinv_from_hist(hist_ref, n):
    deg = hist_ref[0, 0:n, 0:1] + hist_ref[1, 0:n, 0:1] + 1.0
    return lax.rsqrt(deg)


def _mm(a, b):
    return jnp.dot(a, b, preferred_element_type=jnp.float32)


def _tc_mm1(x_ref, w_ref, o_ref):
    o_ref[...] = _mm(x_ref[...], w_ref[...])


def _tc_scale(mm_ref, hist_ref, o_ref):
    o_ref[...] = _dinv_from_hist(hist_ref, mm_ref.shape[0]) * mm_ref[...]


def _agg_sum(agg_ref, n):
    return agg_ref[0, 0:n, :] + agg_ref[1, 0:n, :]


def _tc_layer(agg_ref, hs_ref, hist_ref, b_ref, w_ref, x_ref, hn_ref):
    n = hs_ref.shape[0]
    dinv = _dinv_from_hist(hist_ref, n)
    xl = _elu(dinv * (_agg_sum(agg_ref, n) + hs_ref[...]) + b_ref[...])
    x_ref[...] = xl
    hn_ref[...] = dinv * _mm(xl, w_ref[...])


def _tc_final(x1_ref, x2_ref, agg_ref, hs_ref, hist_ref, b3_ref, batch_ref,
              p1_ref, pb1_ref, g1_ref, bt1_ref, p2_ref, pb2_ref, g2_ref,
              bt2_ref, m1_ref, mb1_ref, g3_ref, bt3_ref, m2_ref, mb2_ref,
              g4_ref, bt4_ref, m3_ref, mb3_ref, o_ref):
    n = hs_ref.shape[0]
    dinv = _dinv_from_hist(hist_ref, n)
    x3 = _elu(dinv * (_agg_sum(agg_ref, n) + hs_ref[...]) + b3_ref[...])

    h = x1_ref.shape[1]
    z1 = _lrelu(_mm(x1_ref[...], p1_ref[0:h]) + _mm(x2_ref[...], p1_ref[h:2 * h])
                + _mm(x3, p1_ref[2 * h:3 * h]) + pb1_ref[...])
    nn = jnp.float32(z1.shape[0])
    mu1 = jnp.sum(z1, axis=0, keepdims=True) / nn
    var1 = jnp.sum(z1 * z1, axis=0, keepdims=True) / nn - mu1 * mu1
    a1 = lax.rsqrt(var1 + 1e-5) * g1_ref[...]
    c1 = bt1_ref[...] - mu1 * a1

    z2 = _lrelu(_mm(z1 * a1 + c1, p2_ref[...]) + pb2_ref[...])
    mu2 = jnp.sum(z2, axis=0, keepdims=True) / nn
    var2 = jnp.sum(z2 * z2, axis=0, keepdims=True) / nn - mu2 * mu2
    a2 = lax.rsqrt(var2 + 1e-5) * g2_ref[...]
    c2 = bt2_ref[...] - mu2 * a2

    g = o_ref.shape[0]
    seg_ids = lax.broadcasted_iota(jnp.int32, (g, 1), 0)
    onehot = (seg_ids == batch_ref[...]).astype(jnp.float32)
    seg = _mm(onehot, z2)
    cnt = jnp.sum(onehot, axis=1, keepdims=True)
    cntm = jnp.maximum(cnt, 1.0)
    pooled = (seg * a2 + c2 * cnt) / cntm

    def bn_small(v, gg, bb):
        gn = jnp.float32(v.shape[0])
        mu = jnp.sum(v, axis=0, keepdims=True) / gn
        var = jnp.sum(v * v, axis=0, keepdims=True) / gn - mu * mu
        return (v - mu) * lax.rsqrt(var + 1e-5) * gg + bb

    m = bn_small(_lrelu(_mm(pooled, m1_ref[...]) + mb1_ref[...]),
                 g3_ref[...], bt3_ref[...])
    m = bn_small(_lrelu(_mm(m, m2_ref[...]) + mb2_ref[...]),
                 g4_ref[...], bt4_ref[...])
    m = _mm(m, m3_ref[...]) + mb3_ref[...]
    mx = jnp.max(m, axis=1, keepdims=True)
    s = m - mx
    o_ref[...] = s - jnp.log(jnp.sum(jnp.exp(s), axis=1, keepdims=True))


def _call(body, out_shape, *args):
    return pl.pallas_call(body, out_shape=out_shape)(*args)


def kernel(x, edge_index, batch, W1, b1, W2, b2, W3, b3, P1, pb1, g1, bt1,
           P2, pb2, g2, bt2, M1, mb1, g3, bt3, M2, mb2, g4, bt4, M3, mb3):
    n, d = x.shape
    e = edge_index.shape[1]
    h = W1.shape[1]
    G = 64
    O = M3.shape[1]
    rows_per_sub = _pad_n(n) // NS

    ei = edge_index.astype(jnp.int32)
    src1d = ei[0]
    dst1d = ei[1]
    zeros_h = jnp.zeros((rows_per_sub, h), jnp.float32)
    batch2d = batch.astype(jnp.int32).reshape(1, n)

    f32 = jnp.float32
    nd = jax.ShapeDtypeStruct((n, h), f32)

    hist = _make_sc_degree(n, e)(zeros_h, dst1d)
    mm1 = _call(_tc_mm1, nd, x, W1)
    h1s = _call(_tc_scale, nd, mm1, hist)

    agg1 = _make_sc_agg(n, e, h)(zeros_h, h1s, src1d, dst1d)
    x1, h2s = _call(_tc_layer, (nd, nd), agg1, h1s, hist,
                    b1.reshape(1, h), W2)
    agg2 = _make_sc_agg(n, e, h)(zeros_h, h2s, src1d, dst1d)
    x2, h3s = _call(_tc_layer, (nd, nd), agg2, h2s, hist,
                    b2.reshape(1, h), W3)
    agg3 = _make_sc_agg(n, e, h)(zeros_h, h3s, src1d, dst1d)

    out = _call(_tc_final, jax.ShapeDtypeStruct((G, O), f32),
                x1, x2, agg3, h3s, hist, b3.reshape(1, h), batch2d,
                P1, pb1.reshape(1, h), g1.reshape(1, h), bt1.reshape(1, h),
                P2, pb2.reshape(1, h), g2.reshape(1, h), bt2.reshape(1, h),
                M1, mb1.reshape(1, h), g3.reshape(1, h), bt3.reshape(1, h),
                M2, mb2.reshape(1, h), g4.reshape(1, h), bt4.reshape(1, h),
                M3, mb3.reshape(1, O))
    return out
